# baseline (speedup 1.0000x reference)
import functools

import jax
import jax.numpy as jnp
from jax.experimental import pallas as pl
from jax.experimental.pallas import tpu as pltpu


def _se_row_kernel(x_ref, w1_ref, w2_ref, o_ref, *, inv_hw):
    xb = x_ref[0]
    pooled = jnp.sum(xb, axis=1, keepdims=True, dtype=jnp.float32) * inv_hw
    h = jnp.dot(w1_ref[...], pooled, preferred_element_type=jnp.float32)
    h = jnp.maximum(h, 0.0)
    s = jax.nn.sigmoid(
        jnp.dot(w2_ref[...], h, preferred_element_type=jnp.float32))
    o_ref[0] = xb * s.astype(o_ref.dtype)


def kernel(x, w1, w2):
    B, C, H, W = x.shape
    HW = H * W
    mid = w1.shape[0]
    itemsize = jnp.dtype(x.dtype).itemsize

    x2 = x.reshape(B, C, HW)

    cost = pl.CostEstimate(
        flops=2 * B * C * HW + 4 * B * C * mid,
        transcendentals=B * C,
        bytes_accessed=2 * B * C * HW * itemsize
        + (w1.size + w2.size) * itemsize,
    )
    out = pl.pallas_call(
        functools.partial(_se_row_kernel, inv_hw=1.0 / HW),
        out_shape=jax.ShapeDtypeStruct((B, C, HW), x.dtype),
        grid=(B,),
        in_specs=[
            pl.BlockSpec((1, C, HW), lambda b: (b, 0, 0)),
            pl.BlockSpec((mid, C), lambda b: (0, 0)),
            pl.BlockSpec((C, mid), lambda b: (0, 0)),
        ],
        out_specs=pl.BlockSpec((1, C, HW), lambda b: (b, 0, 0)),
        compiler_params=pltpu.CompilerParams(
            dimension_semantics=("parallel",),
            vmem_limit_bytes=64 << 20,
        ),
        cost_estimate=cost,
    )(x2, w1, w2)
    return out.reshape(B, C, H, W)
